# Initial kernel scaffold; baseline (speedup 1.0000x reference)
import jax
import jax.numpy as jnp
from jax.experimental import pallas as pl


def kernel(x, pos0, edge_index0, cluster0, edge_index1, cluster1, edge_index2, cluster2, edge_index3, cluster3, edge_index4, cluster4, params):
    raise NotImplementedError("write your pallas kernel here")


# speedup vs baseline: 39.9963x; 39.9963x over previous
import functools

import jax
import jax.numpy as jnp
import numpy as np
from jax.experimental import pallas as pl

N0, N1, N2, N3, N4, NF = 50000, 12500, 3125, 800, 400, 200
K = 5
CONVS = [(1, 16), (19, 16), (19, 24), (27, 32), (35, 40)]
LINS = [19, 19, 27, 35, 43]
NS = [N0, N1, N2, N3, N4]
NXT = [N1, N2, N3, N4, NF]
ES = [800000, 200000, 50000, 12800, 6400]


def _dense_body(x_ref, w_ref, b_ref, o_ref, *, act):
    y = jnp.dot(x_ref[...], w_ref[...], preferred_element_type=jnp.float32)
    y = y + b_ref[...]
    if act == "elu":
        y = jnp.where(y > 0, y, jnp.exp(jnp.minimum(y, 0.0)) - 1.0)
    o_ref[...] = y


def _dense(x, w, b, act="none", block_rows=2048):
    n, cin = x.shape
    cout = w.shape[1]
    nblk = pl.cdiv(n, block_rows)
    npad = nblk * block_rows
    if npad != n:
        x = jnp.pad(x, ((0, npad - n), (0, 0)))
    out = pl.pallas_call(
        functools.partial(_dense_body, act=act),
        grid=(nblk,),
        in_specs=[
            pl.BlockSpec((block_rows, cin), lambda i: (i, jnp.int32(0))),
            pl.BlockSpec((cin, cout), lambda i: (jnp.int32(0), jnp.int32(0))),
            pl.BlockSpec((1, cout), lambda i: (jnp.int32(0), jnp.int32(0))),
        ],
        out_specs=pl.BlockSpec((block_rows, cout), lambda i: (i, jnp.int32(0))),
        out_shape=jax.ShapeDtypeStruct((npad, cout), jnp.float32),
    )(x, w, b.reshape(1, -1))
    return out[:n]


def _segmean(d, s, n):
    tot = jax.ops.segment_sum(d, s, num_segments=n)
    cnt = jax.ops.segment_sum(jnp.ones((d.shape[0], 1), d.dtype), s, num_segments=n)
    return tot / jnp.maximum(cnt, 1.0)


def _segmax(d, s, n):
    m = jax.ops.segment_max(d, s, num_segments=n)
    return jnp.where(jnp.isfinite(m), m, 0.0)


def _inv_conv(p, i, h, pos, src, dst):
    n = pos.shape[0]
    d = pos[dst] - pos[src]
    m = jnp.max(jnp.abs(d)) + 1e-9
    u = 0.5 + d / (2.0 * m)
    closs = jnp.float32(0.0)
    cnt = jax.ops.segment_sum(jnp.ones((src.shape[0], 1), jnp.float32), dst,
                              num_segments=n)
    inv_cnt = 1.0 / jnp.maximum(cnt, 1.0)
    for j in range(2):
        cin, cout = h.shape[1], p[f'c{i}_W{j}'].shape[2]
        coeff = jax.nn.softmax(u @ p[f'c{i}_A{j}'] + p[f'c{i}_Ab{j}'], axis=-1)
        w_flat = jnp.transpose(p[f'c{i}_W{j}'], (1, 0, 2)).reshape(cin, K * cout)
        g = _dense(h, w_flat, jnp.zeros((K * cout,), jnp.float32))
        ge = g[src].reshape(-1, K, cout)
        msg = jnp.einsum('ek,ekd->ed', coeff, ge)
        h = jax.ops.segment_sum(msg, dst, num_segments=n) * inv_cnt
        h = h + p[f'c{i}_b{j}']
        h = jnp.where(h > 0, h, jnp.expm1(h))
        closs = closs + jnp.mean((coeff - 1.0 / K) ** 2)
    return h, closs


def kernel(x, pos0, edge_index0, cluster0, edge_index1, cluster1, edge_index2,
           cluster2, edge_index3, cluster3, edge_index4, cluster4, params):
    eis = [edge_index0, edge_index1, edge_index2, edge_index3, edge_index4]
    cls = [cluster0, cluster1, cluster2, cluster3, cluster4]
    eis = [e.astype(jnp.int32) for e in eis]
    cls = [c.astype(jnp.int32) for c in cls]
    p = jax.tree.map(lambda a: a.astype(jnp.float32), params)
    pos = pos0
    h = x
    closs = jnp.float32(0.0)
    for i in range(5):
        src, dst = eis[i][0], eis[i][1]
        h, cl = _inv_conv(p, i, h, pos, src, dst)
        closs = closs + cl
        h = jnp.concatenate([h, pos], axis=1)
        h = _dense(h, p[f'l{i}_W'], p[f'l{i}_b'], act="elu")
        n = NXT[i]
        if i < 4:
            h = _segmax(h, cls[i], n)
            pos = _segmean(pos, cls[i], n)
    h = _segmax(h, cls[4], NF)
    z = h.reshape(-1, 8 * 43)
    logits = _dense(z, p['fc_W'], p['fc_b'], block_rows=32)
    out = jax.nn.log_softmax(logits, axis=1)
    return out.astype(jnp.float64), closs


# speedup vs baseline: 39.9965x; 1.0000x over previous
import functools

import jax
import jax.numpy as jnp
import numpy as np
from jax.experimental import pallas as pl

N0, N1, N2, N3, N4, NF = 50000, 12500, 3125, 800, 400, 200
K = 5
CONVS = [(1, 16), (19, 16), (19, 24), (27, 32), (35, 40)]
LINS = [19, 19, 27, 35, 43]
NXT = [N1, N2, N3, N4, NF]


def _dense_body(x_ref, w_ref, b_ref, o_ref, *, act):
    y = jnp.dot(x_ref[...], w_ref[...], preferred_element_type=jnp.float32)
    y = y + b_ref[...]
    if act == "elu":
        y = jnp.where(y > 0, y, jnp.exp(jnp.minimum(y, 0.0)) - 1.0)
    o_ref[...] = y


def _dense(x, w, b, act="none", block_rows=2048):
    n, cin = x.shape
    cout = w.shape[1]
    nblk = pl.cdiv(n, block_rows)
    npad = nblk * block_rows
    if npad != n:
        x = jnp.pad(x, ((0, npad - n), (0, 0)))
    out = pl.pallas_call(
        functools.partial(_dense_body, act=act),
        grid=(nblk,),
        in_specs=[
            pl.BlockSpec((block_rows, cin), lambda i: (i, jnp.int32(0))),
            pl.BlockSpec((cin, cout), lambda i: (jnp.int32(0), jnp.int32(0))),
            pl.BlockSpec((1, cout), lambda i: (jnp.int32(0), jnp.int32(0))),
        ],
        out_specs=pl.BlockSpec((block_rows, cout), lambda i: (i, jnp.int32(0))),
        out_shape=jax.ShapeDtypeStruct((npad, cout), jnp.float32),
    )(x, w, b.reshape(1, -1))
    return out[:n]


def _segmean(d, s, n):
    tot = jax.ops.segment_sum(d, s, num_segments=n)
    cnt = jax.ops.segment_sum(jnp.ones((d.shape[0], 1), d.dtype), s, num_segments=n)
    return tot / jnp.maximum(cnt, 1.0)


def _segmax(d, s, n):
    m = jax.ops.segment_max(d, s, num_segments=n)
    return jnp.where(jnp.isfinite(m), m, 0.0)


def _inv_conv(p, i, h, pos, src, dst):
    n = pos.shape[0]
    d = pos[dst] - pos[src]
    m = jnp.max(jnp.abs(d)) + 1e-9
    u = 0.5 + d / (2.0 * m)
    closs = jnp.float32(0.0)
    cnt = jax.ops.segment_sum(jnp.ones((src.shape[0], 1), jnp.float32), dst,
                              num_segments=n)
    inv_cnt = 1.0 / jnp.maximum(cnt, 1.0)
    for j in range(2):
        cin, cout = h.shape[1], p[f'c{i}_W{j}'].shape[2]
        coeff = jax.nn.softmax(u @ p[f'c{i}_A{j}'] + p[f'c{i}_Ab{j}'], axis=-1)
        w_flat = jnp.transpose(p[f'c{i}_W{j}'], (1, 0, 2)).reshape(cin, K * cout)
        g = _dense(h, w_flat, jnp.zeros((K * cout,), jnp.float32))
        ge = g[src].reshape(-1, K, cout)
        msg = jnp.einsum('ek,ekd->ed', coeff, ge)
        h = jax.ops.segment_sum(msg, dst, num_segments=n) * inv_cnt
        h = h + p[f'c{i}_b{j}']
        h = jnp.where(h > 0, h, jnp.exp(jnp.minimum(h, 0.0)) - 1.0)
        closs = closs + jnp.mean((coeff - 1.0 / K) ** 2)
    return h, closs


def kernel(x, pos0, edge_index0, cluster0, edge_index1, cluster1, edge_index2,
           cluster2, edge_index3, cluster3, edge_index4, cluster4, params):
    eis = [edge_index0, edge_index1, edge_index2, edge_index3, edge_index4]
    cls = [cluster0, cluster1, cluster2, cluster3, cluster4]
    eis = [e.astype(jnp.int32) for e in eis]
    cls = [c.astype(jnp.int32) for c in cls]
    p = jax.tree.map(lambda a: a.astype(jnp.float32), params)
    pos = pos0
    h = x
    closs = jnp.float32(0.0)
    for i in range(5):
        src, dst = eis[i][0], eis[i][1]
        h, cl = _inv_conv(p, i, h, pos, src, dst)
        closs = closs + cl
        h = jnp.concatenate([h, pos], axis=1)
        h = _dense(h, p[f'l{i}_W'], p[f'l{i}_b'], act="elu")
        n = NXT[i]
        if i < 4:
            h = _segmax(h, cls[i], n)
            pos = _segmean(pos, cls[i], n)
    h = _segmax(h, cls[4], NF)
    z = h.reshape(-1, 8 * 43)
    logits = _dense(z, p["fc_W"], p["fc_b"], block_rows=32)
    out = jax.nn.log_softmax(logits, axis=1)
    return out.astype(jnp.float64), closs
